# Initial kernel scaffold; baseline (speedup 1.0000x reference)
import jax
import jax.numpy as jnp
from jax.experimental import pallas as pl


def kernel(latents, k):
    raise NotImplementedError("write your pallas kernel here")


# speedup vs baseline: 11.9779x; 11.9779x over previous
import jax
import jax.numpy as jnp
from jax.experimental import pallas as pl

N = 8192
C = 8
K = 512
R = 256

def _select_kernel(x_ref, k_ref, o_ref):
    x = x_ref[...]
    kk = k_ref[...]
    b = jax.lax.bitcast_convert_type(x, jnp.int32)
    key = jnp.where(b < 0, (~b) ^ jnp.int32(-2**31), b)

    lo0 = jnp.full((R, 1), -2**31, jnp.int32)
    hi0 = jnp.full((R, 1), 2**31 - 1, jnp.int32)

    kf = kk.astype(jnp.float32)

    def body(_, carry):
        lo, hi = carry
        mid = (lo >> 1) + (hi >> 1) + (lo & hi & 1)
        gt = jnp.where(key > mid, 1.0, 0.0)
        cnt = jnp.sum(gt, axis=1, keepdims=True)
        go_up = cnt > kf
        lo = jnp.where(go_up, mid + 1, lo)
        hi = jnp.where(go_up, hi, mid)
        return lo, hi

    lo, _ = jax.lax.fori_loop(0, 32, body, (lo0, hi0))
    a = lo
    eq = key == a
    m = jnp.sum(jnp.where(key > a, 1.0, 0.0), axis=1, keepdims=True)
    ii = jax.lax.broadcasted_iota(jnp.int32, (K, K), 0)
    jj = jax.lax.broadcasted_iota(jnp.int32, (K, K), 1)
    tri = jnp.where(ii < jj, 1.0, 0.0)
    eqf = jnp.where(eq, 1.0, 0.0)
    t = jax.lax.dot(eqf, tri, precision=jax.lax.Precision.HIGHEST)
    sel = jnp.logical_and(eq, t == (kf - m))
    o_ref[...] = jnp.where(sel, 1.0, 0.0).astype(jnp.float32)


def kernel(latents, k):
    x = latents.reshape(N * C, K)
    k_rows = jnp.tile(k.astype(jnp.int32), N).reshape(N * C, 1)
    out = pl.pallas_call(
        _select_kernel,
        grid=(N * C // R,),
        in_specs=[
            pl.BlockSpec((R, K), lambda i: (i, 0)),
            pl.BlockSpec((R, 1), lambda i: (i, 0)),
        ],
        out_specs=pl.BlockSpec((R, K), lambda i: (i, 0)),
        out_shape=jax.ShapeDtypeStruct((N * C, K), jnp.float32),
    )(x, k_rows)
    return out.reshape(N, C * K)
